# baseline (speedup 1.0000x reference)
import functools

import jax
import jax.numpy as jnp
from jax import lax
from jax.experimental import pallas as pl
from jax.experimental.pallas import tpu as pltpu
from jax.experimental.pallas import tpu_sc as plsc

N = 10000
E = 320000
H = 64
NC = 2
NS = 16
NW = NC * NS
EPW = E // NW
K = 112
NCH = 90
EPP = NCH * K
NPAD = 10240
RPS = NPAD // NS
HP = 128

_sc_mesh = plsc.VectorSubcoreMesh(core_axis_name="c", subcore_axis_name="s")


@functools.partial(
    pl.kernel,
    out_type=jax.ShapeDtypeStruct((NC, NPAD, HP), jnp.float32),
    mesh=_sc_mesh,
    compiler_params=pltpu.CompilerParams(use_tc_tiling_on_sc=False),
    scratch_types=[
        pltpu.VMEM((NCH, K), jnp.int32),
        pltpu.VMEM((NCH, K), jnp.int32),
        pltpu.VMEM((2, K), jnp.float32),
        pltpu.VMEM((K, HP), jnp.float32),
        pltpu.VMEM((K, HP), jnp.float32),
        pltpu.VMEM_SHARED((NPAD, HP), jnp.float32),
        pltpu.SemaphoreType.DMA,
        pltpu.SemaphoreType.DMA,
        pltpu.SemaphoreType.DMA,
        pltpu.SemaphoreType.DMA,
    ],
)
def _seg_sum_sc(h_hbm, src_hbm, dst_hbm, ew_hbm, out_hbm,
                srcv, dstv, ewc, grows0, grows1, acc,
                gsem0, gsem1, esem0, esem1):
    cid = lax.axis_index("c")
    sid = lax.axis_index("s")
    wid = cid * NS + sid
    grows = (grows0, grows1)
    gsem = (gsem0, gsem1)
    esem = (esem0, esem1)

    @pl.loop(0, K)
    def _(i):
        for j in range(HP // 16):
            grows0[i, pl.ds(j * 16, 16)] = jnp.zeros((16,), jnp.float32)

    pltpu.async_copy(src_hbm.at[wid], srcv, esem0)
    pltpu.async_copy(dst_hbm.at[wid], dstv, esem1)

    @pl.loop(0, RPS // K)
    def _(t):
        pltpu.async_copy(grows0, acc.at[pl.ds(sid * RPS + t * K, K)], gsem0)

    @pl.loop(0, RPS // K)
    def _(t):
        pltpu.make_async_copy(grows0, acc.at[pl.ds(0, K)], gsem0).wait()

    pltpu.make_async_copy(src_hbm.at[0], srcv, esem0).wait()
    pltpu.make_async_copy(dst_hbm.at[0], dstv, esem1).wait()

    plsc.subcore_barrier()

    def start(ci, b):
        pltpu.async_copy(h_hbm.at[srcv.at[ci]], grows[b], gsem[b])
        pltpu.async_copy(ew_hbm.at[wid, ci], ewc.at[b], esem[b])

    def finish(ci, b):
        pltpu.make_async_copy(h_hbm.at[srcv.at[0]], grows[b], gsem[b]).wait()
        pltpu.make_async_copy(ew_hbm.at[wid, 0], ewc.at[b], esem[b]).wait()

        @pl.loop(0, K // 16)
        def _(gidx):
            ws = ewc[b, pl.ds(gidx * 16, 16)]
            for i in range(16):
                s = ws[i]
                row = gidx * 16 + i
                for j in range(H // 16):
                    grows[b][row, pl.ds(j * 16, 16)] = (
                        grows[b][row, pl.ds(j * 16, 16)] * s)

        pltpu.sync_copy(grows[b], acc.at[dstv.at[ci]], add=True)

    start(0, 0)

    @pl.loop(0, NCH // 2 - 1)
    def _(t):
        c = 2 * t
        start(c + 1, 1)
        finish(c, 0)
        start(c + 2, 0)
        finish(c + 1, 1)

    start(NCH - 1, 1)
    finish(NCH - 2, 0)
    finish(NCH - 1, 1)

    plsc.subcore_barrier()

    pltpu.sync_copy(acc.at[pl.ds(sid * RPS, RPS)],
                    out_hbm.at[cid, pl.ds(sid * RPS, RPS)])


def _mm_body(x_ref, w_ref, o_ref):
    o_ref[...] = jnp.dot(x_ref[...], w_ref[...],
                         preferred_element_type=jnp.float32)


def _mm(xin, W, bn=2000):
    n, f = xin.shape
    g = W.shape[1]
    return pl.pallas_call(
        _mm_body,
        grid=(n // bn,),
        in_specs=[pl.BlockSpec((bn, f), lambda i: (i, 0)),
                  pl.BlockSpec((f, g), lambda i: (0, 0))],
        out_specs=pl.BlockSpec((bn, g), lambda i: (i, 0)),
        out_shape=jax.ShapeDtypeStruct((n, g), jnp.float32),
    )(xin, W)


def _gates_body(final, xw_ref, p_ref, c_ref, Wh_ref, b_ref,
                wci_ref, wcf_ref, wco_ref, wlin_ref, blin_ref, *out_refs):
    agg = (p_ref[0] + p_ref[1])[:, :H]
    g = (xw_ref[...]
         + jnp.dot(agg, Wh_ref[...], preferred_element_type=jnp.float32)
         + b_ref[...])
    c = c_ref[...]
    gate_i = jax.nn.sigmoid(g[:, 0:H] + wci_ref[...] * c)
    gate_f = jax.nn.sigmoid(g[:, H:2 * H] + wcf_ref[...] * c)
    c_new = gate_f * c + gate_i * jnp.tanh(g[:, 2 * H:3 * H])
    gate_o = jax.nn.sigmoid(g[:, 3 * H:4 * H] + wco_ref[...] * c_new)
    h_new = gate_o * jnp.tanh(c_new)
    out_refs[0][...] = h_new
    out_refs[1][...] = c_new
    if not final:
        out_refs[2][...] = jnp.concatenate(
            [h_new, jnp.zeros_like(h_new)], axis=1)
    if final:
        y = jnp.maximum(h_new, 0.0) * wlin_ref[...]
        out_refs[2][...] = jnp.sum(y, axis=1, keepdims=True) + blin_ref[...]


def _gates(xw, p, c, Wh, b, wci, wcf, wco, wlin, blin, final, bn=2000):
    n = xw.shape[0]
    out_shapes = [jax.ShapeDtypeStruct((n, H), jnp.float32),
                  jax.ShapeDtypeStruct((n, H), jnp.float32)]
    out_specs = [pl.BlockSpec((bn, H), lambda i: (i, 0)),
                 pl.BlockSpec((bn, H), lambda i: (i, 0))]
    if final:
        out_shapes.append(jax.ShapeDtypeStruct((n, 1), jnp.float32))
        out_specs.append(pl.BlockSpec((bn, 1), lambda i: (i, 0)))
    else:
        out_shapes.append(jax.ShapeDtypeStruct((n, HP), jnp.float32))
        out_specs.append(pl.BlockSpec((bn, HP), lambda i: (i, 0)))
    return pl.pallas_call(
        functools.partial(_gates_body, final),
        grid=(n // bn,),
        in_specs=[
            pl.BlockSpec((bn, 4 * H), lambda i: (i, 0)),
            pl.BlockSpec((NC, bn, HP), lambda i: (0, i, 0)),
            pl.BlockSpec((bn, H), lambda i: (i, 0)),
            pl.BlockSpec((H, 4 * H), lambda i: (0, 0)),
            pl.BlockSpec((1, 4 * H), lambda i: (0, 0)),
            pl.BlockSpec((1, H), lambda i: (0, 0)),
            pl.BlockSpec((1, H), lambda i: (0, 0)),
            pl.BlockSpec((1, H), lambda i: (0, 0)),
            pl.BlockSpec((1, H), lambda i: (0, 0)),
            pl.BlockSpec((1, 1), lambda i: (0, 0)),
        ],
        out_specs=out_specs,
        out_shape=out_shapes,
    )(xw, p, c, Wh, b.reshape(1, 4 * H), wci.reshape(1, H),
      wcf.reshape(1, H), wco.reshape(1, H), wlin.reshape(1, H),
      blin.reshape(1, 1))


def kernel(x, edge_index, edge_weight, h, c,
           Wx0, Wh0, b0, wci0, wcf0, wco0,
           Wx1, Wh1, b1, wci1, wcf1, wco1, Wlin, blin):
    pad_w = ((0, 0), (0, EPP - EPW))
    src = jnp.pad(edge_index[0].reshape(NW, EPW), pad_w).reshape(NW, NCH, K)
    dst = jnp.pad(edge_index[1].reshape(NW, EPW), pad_w,
                  constant_values=N).reshape(NW, NCH, K)
    ew = jnp.pad(edge_weight.reshape(NW, EPW), pad_w).reshape(NW, NCH, K)

    h_pad = jnp.pad(h, ((0, 0), (0, HP - H)))

    p0 = _seg_sum_sc(h_pad, src, dst, ew)
    xw0 = _mm(x, Wx0)
    h0, c0, h0_pad = _gates(xw0, p0, c, Wh0, b0, wci0, wcf0, wco0, Wlin, blin,
                            final=False)

    p1 = _seg_sum_sc(h0_pad, src, dst, ew)
    xw1 = _mm(h0, Wx1)
    h1, c1, out = _gates(xw1, p1, c0, Wh1, b1, wci1, wcf1, wco1, Wlin, blin,
                         final=True)
    return (out, h1, c1)


# speedup vs baseline: 1.5170x; 1.5170x over previous
import functools

import jax
import jax.numpy as jnp
from jax import lax
from jax.experimental import pallas as pl
from jax.experimental.pallas import tpu as pltpu
from jax.experimental.pallas import tpu_sc as plsc

N = 10000
E = 320000
H = 64
NC = 2
NS = 16
NW = NC * NS
EPW = E // NW
K = 80
NCH = EPW // K
NPAD = 10240
RPS = NPAD // NS
HP = 128

_sc_mesh = plsc.VectorSubcoreMesh(core_axis_name="c", subcore_axis_name="s")


@functools.partial(
    pl.kernel,
    out_type=jax.ShapeDtypeStruct((NC, NPAD, HP), jnp.float32),
    mesh=_sc_mesh,
    compiler_params=pltpu.CompilerParams(use_tc_tiling_on_sc=False),
    scratch_types=[
        pltpu.VMEM((NCH, K), jnp.int32),
        pltpu.VMEM((NCH, K), jnp.int32),
        pltpu.VMEM((2, K), jnp.float32),
        pltpu.VMEM((K, HP), jnp.float32),
        pltpu.VMEM((K, HP), jnp.float32),
        pltpu.VMEM_SHARED((NPAD, HP), jnp.float32),
        pltpu.SemaphoreType.DMA,
        pltpu.SemaphoreType.DMA,
        pltpu.SemaphoreType.DMA,
        pltpu.SemaphoreType.DMA,
    ],
)
def _seg_sum_sc(h_hbm, src_hbm, dst_hbm, ew_hbm, out_hbm,
                srcv, dstv, ewc, grows0, grows1, acc,
                gsem0, gsem1, esem0, esem1):
    cid = lax.axis_index("c")
    sid = lax.axis_index("s")
    wid = cid * NS + sid
    grows = (grows0, grows1)
    gsem = (gsem0, gsem1)
    esem = (esem0, esem1)

    @pl.loop(0, K)
    def _(i):
        for j in range(HP // 16):
            grows0[i, pl.ds(j * 16, 16)] = jnp.zeros((16,), jnp.float32)

    pltpu.async_copy(src_hbm.at[wid], srcv, esem0)
    pltpu.async_copy(dst_hbm.at[wid], dstv, esem1)

    @pl.loop(0, RPS // K)
    def _(t):
        pltpu.async_copy(grows0, acc.at[pl.ds(sid * RPS + t * K, K)], gsem0)

    @pl.loop(0, RPS // K)
    def _(t):
        pltpu.make_async_copy(grows0, acc.at[pl.ds(0, K)], gsem0).wait()

    pltpu.make_async_copy(src_hbm.at[0], srcv, esem0).wait()
    pltpu.make_async_copy(dst_hbm.at[0], dstv, esem1).wait()

    plsc.subcore_barrier()

    def start(ci, b):
        pltpu.async_copy(h_hbm.at[srcv.at[ci]], grows[b], gsem[b])
        pltpu.async_copy(ew_hbm.at[wid, ci], ewc.at[b], esem[b])

    def finish(ci, b):
        pltpu.make_async_copy(h_hbm.at[srcv.at[0]], grows[b], gsem[b]).wait()
        pltpu.make_async_copy(ew_hbm.at[wid, 0], ewc.at[b], esem[b]).wait()

        @pl.loop(0, K // 16)
        def _(gidx):
            ws = ewc[b, pl.ds(gidx * 16, 16)]
            for i in range(16):
                s = ws[i]
                row = gidx * 16 + i
                for j in range(H // 16):
                    grows[b][row, pl.ds(j * 16, 16)] = (
                        grows[b][row, pl.ds(j * 16, 16)] * s)

        pltpu.sync_copy(grows[b], acc.at[dstv.at[ci]], add=True)

    start(0, 0)

    @pl.loop(0, (NCH - 1) // 2)
    def _(t):
        c = 2 * t
        start(c + 1, 1)
        finish(c, 0)
        start(c + 2, 0)
        finish(c + 1, 1)

    finish(NCH - 1, 0)

    plsc.subcore_barrier()

    pltpu.sync_copy(acc.at[pl.ds(sid * RPS, RPS)],
                    out_hbm.at[cid, pl.ds(sid * RPS, RPS)])


def _mm_body(x_ref, w_ref, o_ref):
    o_ref[...] = jnp.dot(x_ref[...], w_ref[...],
                         preferred_element_type=jnp.float32)


def _mm(xin, W, bn=2000):
    n, f = xin.shape
    g = W.shape[1]
    return pl.pallas_call(
        _mm_body,
        grid=(n // bn,),
        in_specs=[pl.BlockSpec((bn, f), lambda i: (i, 0)),
                  pl.BlockSpec((f, g), lambda i: (0, 0))],
        out_specs=pl.BlockSpec((bn, g), lambda i: (i, 0)),
        out_shape=jax.ShapeDtypeStruct((n, g), jnp.float32),
    )(xin, W)


def _gates_body(final, xw_ref, p_ref, c_ref, Wh_ref, b_ref,
                wci_ref, wcf_ref, wco_ref, wlin_ref, blin_ref, *out_refs):
    agg = (p_ref[0] + p_ref[1])[:, :H]
    g = (xw_ref[...]
         + jnp.dot(agg, Wh_ref[...], preferred_element_type=jnp.float32)
         + b_ref[...])
    c = c_ref[...]
    gate_i = jax.nn.sigmoid(g[:, 0:H] + wci_ref[...] * c)
    gate_f = jax.nn.sigmoid(g[:, H:2 * H] + wcf_ref[...] * c)
    c_new = gate_f * c + gate_i * jnp.tanh(g[:, 2 * H:3 * H])
    gate_o = jax.nn.sigmoid(g[:, 3 * H:4 * H] + wco_ref[...] * c_new)
    h_new = gate_o * jnp.tanh(c_new)
    out_refs[0][...] = h_new
    out_refs[1][...] = c_new
    if not final:
        out_refs[2][...] = jnp.concatenate(
            [h_new, jnp.zeros_like(h_new)], axis=1)
    if final:
        y = jnp.maximum(h_new, 0.0) * wlin_ref[...]
        out_refs[2][...] = jnp.sum(y, axis=1, keepdims=True) + blin_ref[...]


def _gates(xw, p, c, Wh, b, wci, wcf, wco, wlin, blin, final, bn=2000):
    n = xw.shape[0]
    out_shapes = [jax.ShapeDtypeStruct((n, H), jnp.float32),
                  jax.ShapeDtypeStruct((n, H), jnp.float32)]
    out_specs = [pl.BlockSpec((bn, H), lambda i: (i, 0)),
                 pl.BlockSpec((bn, H), lambda i: (i, 0))]
    if final:
        out_shapes.append(jax.ShapeDtypeStruct((n, 1), jnp.float32))
        out_specs.append(pl.BlockSpec((bn, 1), lambda i: (i, 0)))
    else:
        out_shapes.append(jax.ShapeDtypeStruct((n, HP), jnp.float32))
        out_specs.append(pl.BlockSpec((bn, HP), lambda i: (i, 0)))
    return pl.pallas_call(
        functools.partial(_gates_body, final),
        grid=(n // bn,),
        in_specs=[
            pl.BlockSpec((bn, 4 * H), lambda i: (i, 0)),
            pl.BlockSpec((NC, bn, HP), lambda i: (0, i, 0)),
            pl.BlockSpec((bn, H), lambda i: (i, 0)),
            pl.BlockSpec((H, 4 * H), lambda i: (0, 0)),
            pl.BlockSpec((1, 4 * H), lambda i: (0, 0)),
            pl.BlockSpec((1, H), lambda i: (0, 0)),
            pl.BlockSpec((1, H), lambda i: (0, 0)),
            pl.BlockSpec((1, H), lambda i: (0, 0)),
            pl.BlockSpec((1, H), lambda i: (0, 0)),
            pl.BlockSpec((1, 1), lambda i: (0, 0)),
        ],
        out_specs=out_specs,
        out_shape=out_shapes,
    )(xw, p, c, Wh, b.reshape(1, 4 * H), wci.reshape(1, H),
      wcf.reshape(1, H), wco.reshape(1, H), wlin.reshape(1, H),
      blin.reshape(1, 1))


def kernel(x, edge_index, edge_weight, h, c,
           Wx0, Wh0, b0, wci0, wcf0, wco0,
           Wx1, Wh1, b1, wci1, wcf1, wco1, Wlin, blin):
    src = edge_index[0].reshape(NW, NCH, K)
    dst = edge_index[1].reshape(NW, NCH, K)
    ew = edge_weight.reshape(NW, NCH, K)

    h_pad = jnp.pad(h, ((0, 0), (0, HP - H)))

    p0 = _seg_sum_sc(h_pad, src, dst, ew)
    xw0 = _mm(x, Wx0)
    h0, c0, h0_pad = _gates(xw0, p0, c, Wh0, b0, wci0, wcf0, wco0, Wlin, blin,
                            final=False)

    p1 = _seg_sum_sc(h0_pad, src, dst, ew)
    xw1 = _mm(h0, Wx1)
    h1, c1, out = _gates(xw1, p1, c0, Wh1, b1, wci1, wcf1, wco1, Wlin, blin,
                         final=True)
    return (out, h1, c1)
